# baseline (speedup 1.0000x reference)
import functools

import jax
import jax.numpy as jnp
from jax import lax
from jax.experimental import pallas as pl
from jax.experimental.pallas import tpu as pltpu
from jax.experimental.pallas import tpu_sc as plsc


def _sc_relpos_copy(pe, n, off):
    V, D = pe.shape
    info = plsc.get_sparse_core_info()
    NC, NS = info.num_cores, info.num_subcores
    NW = NC * NS
    assert n % NW == 0
    rows_per_w = n // NW

    C = 32
    NBUF = 4
    NCH = n // C
    K = rows_per_w * NCH
    assert K % NBUF == 0

    lo = ((off - n + 1) // 8) * 8
    P = jnp.stack([lax.dynamic_slice_in_dim(pe, lo + d, 2 * n) for d in range(8)])

    mesh = plsc.VectorSubcoreMesh(core_axis_name="c", subcore_axis_name="s")

    @functools.partial(
        pl.kernel,
        out_type=jax.ShapeDtypeStruct((n, n, D), jnp.float32),
        mesh=mesh,
        scratch_types=(
            [pltpu.VMEM((C, D), jnp.float32)] * NBUF
            + [pltpu.SemaphoreType.DMA] * (2 * NBUF)
        ),
    )
    def k(p_hbm, out_hbm, *scratch):
        bufs = scratch[:NBUF]
        slds = scratch[NBUF:2 * NBUF]
        ssts = scratch[2 * NBUF:]
        wid = lax.axis_index("s") * NC + lax.axis_index("c")
        i0 = wid * rows_per_w

        def src_dst(step):
            r = lax.div(step, NCH)
            jc = lax.rem(step, NCH) * C
            i = i0 + r
            s = off - i
            d = lax.rem(s, 8)
            a = pl.multiple_of(s - d - lo + jc, 8)
            return p_hbm.at[d, pl.ds(a, C)], out_hbm.at[i, pl.ds(jc, C)]

        def body(g, carry):
            for b in range(NBUF):
                step = NBUF * g + b
                src, dst = src_dst(step)

                @pl.when(g >= 1)
                def _():
                    pltpu.make_async_copy(src, bufs[b], ssts[b]).wait()

                pltpu.async_copy(src, bufs[b], slds[b]).wait()
                pltpu.async_copy(bufs[b], dst, ssts[b])
            return carry

        lax.fori_loop(0, K // NBUF, body, 0)
        src0, _ = src_dst(0)
        for b in range(NBUF):
            pltpu.make_async_copy(src0, bufs[b], ssts[b]).wait()

    return k(P)


def kernel(x, q_len, pe):
    n = x.shape[1]
    V = pe.shape[0]
    off = (V + 1) // 2 - 1
    return _sc_relpos_copy(pe, n, off)


# speedup vs baseline: 1.2766x; 1.2766x over previous
import functools

import jax
import jax.numpy as jnp
from jax import lax
from jax.experimental import pallas as pl
from jax.experimental.pallas import tpu as pltpu
from jax.experimental.pallas import tpu_sc as plsc


def _sc_relpos_copy(pe, n, off):
    V, D = pe.shape
    info = plsc.get_sparse_core_info()
    NC, NS = info.num_cores, info.num_subcores
    NW = NC * NS
    assert n % NW == 0
    rows_per_w = n // NW

    C = 64
    NBUF = 2
    NCH = n // C
    NPAIR = rows_per_w // 2
    K = NPAIR * NCH
    assert K % NBUF == 0

    lo = ((off - n + 1) // 8) * 8 - 8
    P = jnp.stack(
        [lax.dynamic_slice_in_dim(pe, lo + d, 2 * n + 8) for d in range(8)]
    )

    mesh = plsc.VectorSubcoreMesh(core_axis_name="c", subcore_axis_name="s")

    @functools.partial(
        pl.kernel,
        out_type=jax.ShapeDtypeStruct((n, n, D), jnp.float32),
        mesh=mesh,
        scratch_types=(
            [pltpu.VMEM((C + 8, D), jnp.float32)] * NBUF
            + [pltpu.SemaphoreType.DMA] * (2 * NBUF)
        ),
    )
    def k(p_hbm, out_hbm, *scratch):
        bufs = scratch[:NBUF]
        slds = scratch[NBUF:2 * NBUF]
        ssts = scratch[2 * NBUF:]
        wid = lax.axis_index("s") * NC + lax.axis_index("c")
        i0 = wid * rows_per_w

        def plan(step):
            r = lax.div(step, NCH)
            jc = lax.rem(step, NCH) * C
            i = i0 + r
            s = off - i
            d = lax.rem(s, 8)
            a = pl.multiple_of(s - d - lo + jc - 8, 8)
            src = p_hbm.at[d, pl.ds(a, C + 8)]
            dst_hi = out_hbm.at[i, pl.ds(jc, C)]
            dst_lo = out_hbm.at[i + 8, pl.ds(jc, C)]
            return src, dst_hi, dst_lo

        def body(g, carry):
            for b in range(NBUF):
                step = NBUF * g + b
                src, dst_hi, dst_lo = plan(step)

                @pl.when(g >= 1)
                def _():
                    for _ in range(2):
                        pltpu.make_async_copy(
                            src, bufs[b].at[pl.ds(0, C)], ssts[b]
                        ).wait()

                pltpu.async_copy(src, bufs[b], slds[b]).wait()
                pltpu.async_copy(bufs[b].at[pl.ds(8, C)], dst_hi, ssts[b])
                pltpu.async_copy(bufs[b].at[pl.ds(0, C)], dst_lo, ssts[b])
            return carry

        lax.fori_loop(0, K // NBUF, body, 0)
        src0, _, _ = plan(0)
        for b in range(NBUF):
            for _ in range(2):
                pltpu.make_async_copy(
                    src0, bufs[b].at[pl.ds(0, C)], ssts[b]
                ).wait()

    return k(P)


def kernel(x, q_len, pe):
    n = x.shape[1]
    V = pe.shape[0]
    off = (V + 1) // 2 - 1
    return _sc_relpos_copy(pe, n, off)


# speedup vs baseline: 1.3947x; 1.0925x over previous
import functools

import jax
import jax.numpy as jnp
from jax import lax
from jax.experimental import pallas as pl
from jax.experimental.pallas import tpu as pltpu
from jax.experimental.pallas import tpu_sc as plsc


def _sc_relpos_copy(pe, n, off):
    V, D = pe.shape
    info = plsc.get_sparse_core_info()
    NC, NS = info.num_cores, info.num_subcores
    NW = NC * NS
    assert n % NW == 0
    rows_per_w = n // NW

    C = 32
    GROUP = 4
    PAD = 8 * (GROUP - 1)
    NBUF = 2
    NCH = n // C
    NGRP = rows_per_w // GROUP
    K = NGRP * NCH
    assert K % NBUF == 0

    lo = ((off - n + 1) // 8) * 8 - PAD
    P = jnp.stack(
        [lax.dynamic_slice_in_dim(pe, lo + d, 2 * n + PAD) for d in range(8)]
    )

    mesh = plsc.VectorSubcoreMesh(core_axis_name="c", subcore_axis_name="s")

    @functools.partial(
        pl.kernel,
        out_type=jax.ShapeDtypeStruct((n, n, D), jnp.float32),
        mesh=mesh,
        scratch_types=(
            [pltpu.VMEM((C + PAD, D), jnp.float32)] * NBUF
            + [pltpu.SemaphoreType.DMA] * (2 * NBUF)
        ),
    )
    def k(p_hbm, out_hbm, *scratch):
        bufs = scratch[:NBUF]
        slds = scratch[NBUF:2 * NBUF]
        ssts = scratch[2 * NBUF:]
        wid = lax.axis_index("s") * NC + lax.axis_index("c")
        blk = lax.div(wid, 2) * (2 * rows_per_w)
        r0 = lax.rem(wid, 2) * NGRP

        def plan(step):
            grp = lax.div(step, NCH)
            jc = lax.rem(step, NCH) * C
            i = blk + r0 + grp
            s = off - i
            d = lax.rem(s, 8)
            a = pl.multiple_of(s - d - lo + jc - PAD, 8)
            src = p_hbm.at[d, pl.ds(a, C + PAD)]
            dsts = [out_hbm.at[i + 8 * kk, pl.ds(jc, C)] for kk in range(GROUP)]
            return src, dsts

        def body(g, carry):
            for b in range(NBUF):
                step = NBUF * g + b
                src, dsts = plan(step)

                @pl.when(g >= 1)
                def _():
                    for _ in range(GROUP):
                        pltpu.make_async_copy(
                            src, bufs[b].at[pl.ds(0, C)], ssts[b]
                        ).wait()

                pltpu.async_copy(src, bufs[b], slds[b]).wait()
                for kk in range(GROUP):
                    pltpu.async_copy(
                        bufs[b].at[pl.ds(PAD - 8 * kk, C)], dsts[kk], ssts[b]
                    )
            return carry

        lax.fori_loop(0, K // NBUF, body, 0)
        src0, _ = plan(0)
        for b in range(NBUF):
            for _ in range(GROUP):
                pltpu.make_async_copy(
                    src0, bufs[b].at[pl.ds(0, C)], ssts[b]
                ).wait()

    return k(P)


def kernel(x, q_len, pe):
    n = x.shape[1]
    V = pe.shape[0]
    off = (V + 1) // 2 - 1
    return _sc_relpos_copy(pe, n, off)


# speedup vs baseline: 1.4668x; 1.0517x over previous
import functools

import jax
import jax.numpy as jnp
from jax import lax
from jax.experimental import pallas as pl
from jax.experimental.pallas import tpu as pltpu
from jax.experimental.pallas import tpu_sc as plsc


def _sc_relpos_copy(pe, n, off):
    V, D = pe.shape
    info = plsc.get_sparse_core_info()
    NC, NS = info.num_cores, info.num_subcores
    NW = NC * NS
    assert n % NW == 0
    rows_per_w = n // NW

    C = 64
    GROUP = 4
    PAD = 8 * (GROUP - 1)
    DS = 2
    D2 = D // DS
    NBUF = 2
    NCH = (n // C) * DS
    NGRP = rows_per_w // GROUP
    K = NGRP * NCH
    assert K % NBUF == 0

    lo = ((off - n + 1) // 8) * 8 - PAD
    P = jnp.stack(
        [lax.dynamic_slice_in_dim(pe, lo + d, 2 * n + PAD) for d in range(8)]
    )

    mesh = plsc.VectorSubcoreMesh(core_axis_name="c", subcore_axis_name="s")

    @functools.partial(
        pl.kernel,
        out_type=jax.ShapeDtypeStruct((n, n, D), jnp.float32),
        mesh=mesh,
        scratch_types=(
            [pltpu.VMEM((C + PAD, D2), jnp.float32)] * NBUF
            + [pltpu.SemaphoreType.DMA] * (2 * NBUF)
        ),
    )
    def k(p_hbm, out_hbm, *scratch):
        bufs = scratch[:NBUF]
        slds = scratch[NBUF:2 * NBUF]
        ssts = scratch[2 * NBUF:]
        wid = lax.axis_index("s") * NC + lax.axis_index("c")
        blk = lax.div(wid, 2) * (2 * rows_per_w)
        r0 = lax.rem(wid, 2) * NGRP

        def plan(step):
            grp = lax.div(step, NCH)
            rem = lax.rem(step, NCH)
            jc = lax.div(rem, DS) * C
            h = lax.rem(rem, DS) * D2
            i = blk + r0 + grp
            s = off - i
            d = lax.rem(s, 8)
            a = pl.multiple_of(s - d - lo + jc - PAD, 8)
            hh = pl.multiple_of(h, 128)
            src = p_hbm.at[d, pl.ds(a, C + PAD), pl.ds(hh, D2)]
            dsts = [
                out_hbm.at[i + 8 * kk, pl.ds(jc, C), pl.ds(hh, D2)]
                for kk in range(GROUP)
            ]
            return src, dsts

        def body(g, carry):
            for b in range(NBUF):
                step = NBUF * g + b
                src, dsts = plan(step)

                @pl.when(g >= 1)
                def _():
                    for _ in range(GROUP):
                        pltpu.make_async_copy(
                            src, bufs[b].at[pl.ds(0, C)], ssts[b]
                        ).wait()

                pltpu.async_copy(src, bufs[b], slds[b]).wait()
                for kk in range(GROUP):
                    pltpu.async_copy(
                        bufs[b].at[pl.ds(PAD - 8 * kk, C)], dsts[kk], ssts[b]
                    )
            return carry

        lax.fori_loop(0, K // NBUF, body, 0)
        src0, _ = plan(0)
        for b in range(NBUF):
            for _ in range(GROUP):
                pltpu.make_async_copy(
                    src0, bufs[b].at[pl.ds(0, C)], ssts[b]
                ).wait()

    return k(P)


def kernel(x, q_len, pe):
    n = x.shape[1]
    V = pe.shape[0]
    off = (V + 1) // 2 - 1
    return _sc_relpos_copy(pe, n, off)


# speedup vs baseline: 1.6167x; 1.1022x over previous
import functools

import jax
import jax.numpy as jnp
from jax import lax
from jax.experimental import pallas as pl
from jax.experimental.pallas import tpu as pltpu
from jax.experimental.pallas import tpu_sc as plsc


def _sc_relpos_copy(pe, n, off):
    V, D = pe.shape
    info = plsc.get_sparse_core_info()
    NC, NS = info.num_cores, info.num_subcores
    NW = NC * NS
    assert n % NW == 0
    rows_per_w = n // NW

    C = 64
    GROUP = 8
    PAD = 8 * (GROUP - 1)
    DS = 2
    D2 = D // DS
    NBUF = 2
    NCH = (n // C) * DS
    NGRP = rows_per_w // GROUP
    K = NGRP * NCH
    assert K % NBUF == 0

    lo = ((off - n + 1) // 8) * 8 - PAD
    P = jnp.stack(
        [lax.dynamic_slice_in_dim(pe, lo + d, 2 * n + PAD) for d in range(8)]
    )

    mesh = plsc.VectorSubcoreMesh(core_axis_name="c", subcore_axis_name="s")

    @functools.partial(
        pl.kernel,
        out_type=jax.ShapeDtypeStruct((n, n, D), jnp.float32),
        mesh=mesh,
        scratch_types=(
            [pltpu.VMEM((C + PAD, D2), jnp.float32)] * NBUF
            + [pltpu.SemaphoreType.DMA] * (2 * NBUF)
        ),
    )
    def k(p_hbm, out_hbm, *scratch):
        bufs = scratch[:NBUF]
        slds = scratch[NBUF:2 * NBUF]
        ssts = scratch[2 * NBUF:]
        wid = lax.axis_index("s") * NC + lax.axis_index("c")
        wpb = 8 * GROUP // rows_per_w
        blk = lax.div(wid, wpb) * (8 * GROUP)
        r0 = lax.rem(wid, wpb) * NGRP

        def plan(step):
            grp = lax.div(step, NCH)
            rem = lax.rem(step, NCH)
            jc = lax.div(rem, DS) * C
            h = lax.rem(rem, DS) * D2
            i = blk + r0 + grp
            s = off - i
            d = lax.rem(s, 8)
            a = pl.multiple_of(s - d - lo + jc - PAD, 8)
            hh = pl.multiple_of(h, 128)
            src = p_hbm.at[d, pl.ds(a, C + PAD), pl.ds(hh, D2)]
            dsts = [
                out_hbm.at[i + 8 * kk, pl.ds(jc, C), pl.ds(hh, D2)]
                for kk in range(GROUP)
            ]
            return src, dsts

        def body(g, carry):
            for b in range(NBUF):
                step = NBUF * g + b
                src, dsts = plan(step)

                @pl.when(g >= 1)
                def _():
                    for _ in range(GROUP):
                        pltpu.make_async_copy(
                            src, bufs[b].at[pl.ds(0, C)], ssts[b]
                        ).wait()

                pltpu.async_copy(src, bufs[b], slds[b]).wait()
                for kk in range(GROUP):
                    pltpu.async_copy(
                        bufs[b].at[pl.ds(PAD - 8 * kk, C)], dsts[kk], ssts[b]
                    )
            return carry

        lax.fori_loop(0, K // NBUF, body, 0)
        src0, _ = plan(0)
        for b in range(NBUF):
            for _ in range(GROUP):
                pltpu.make_async_copy(
                    src0, bufs[b].at[pl.ds(0, C)], ssts[b]
                ).wait()

    return k(P)


def kernel(x, q_len, pe):
    n = x.shape[1]
    V = pe.shape[0]
    off = (V + 1) // 2 - 1
    return _sc_relpos_copy(pe, n, off)


# speedup vs baseline: 1.8984x; 1.1742x over previous
import functools

import jax
import jax.numpy as jnp
from jax import lax
from jax.experimental import pallas as pl
from jax.experimental.pallas import tpu as pltpu
from jax.experimental.pallas import tpu_sc as plsc


def _sc_relpos_copy(pe, n, off):
    V, D = pe.shape
    info = plsc.get_sparse_core_info()
    NC, NS = info.num_cores, info.num_subcores
    NW = NC * NS
    assert n % NW == 0
    rows_per_w = n // NW

    C = 128
    GROUP = 16
    PAD = 8 * (GROUP - 1)
    DS = 3
    D2 = D // DS
    NBUF = 2
    NCH = (n // C) * DS
    NGRP = rows_per_w // GROUP
    K = NGRP * NCH
    assert K % NBUF == 0

    lo = ((off - n + 1) // 8) * 8 - PAD
    P = jnp.stack(
        [lax.dynamic_slice_in_dim(pe, lo + d, 2 * n + PAD) for d in range(8)]
    )

    mesh = plsc.VectorSubcoreMesh(core_axis_name="c", subcore_axis_name="s")

    @functools.partial(
        pl.kernel,
        out_type=jax.ShapeDtypeStruct((n, n, D), jnp.float32),
        mesh=mesh,
        scratch_types=(
            [pltpu.VMEM((C + PAD, D2), jnp.float32)] * NBUF
            + [pltpu.SemaphoreType.DMA] * (2 * NBUF)
        ),
    )
    def k(p_hbm, out_hbm, *scratch):
        bufs = scratch[:NBUF]
        slds = scratch[NBUF:2 * NBUF]
        ssts = scratch[2 * NBUF:]
        wid = lax.axis_index("s") * NC + lax.axis_index("c")
        wpb = 8 * GROUP // rows_per_w
        blk = lax.div(wid, wpb) * (8 * GROUP)
        r0 = lax.rem(wid, wpb) * NGRP

        def plan(step):
            grp = lax.div(step, NCH)
            rem = lax.rem(step, NCH)
            jc = lax.div(rem, DS) * C
            h = lax.rem(rem, DS) * D2
            i = blk + r0 + grp
            s = off - i
            d = lax.rem(s, 8)
            a = pl.multiple_of(s - d - lo + jc - PAD, 8)
            hh = pl.multiple_of(h, 128)
            src = p_hbm.at[d, pl.ds(a, C + PAD), pl.ds(hh, D2)]
            dsts = [
                out_hbm.at[i + 8 * kk, pl.ds(jc, C), pl.ds(hh, D2)]
                for kk in range(GROUP)
            ]
            return src, dsts

        def body(g, carry):
            for b in range(NBUF):
                step = NBUF * g + b
                src, dsts = plan(step)

                @pl.when(g >= 1)
                def _():
                    for _ in range(GROUP):
                        pltpu.make_async_copy(
                            src, bufs[b].at[pl.ds(0, C)], ssts[b]
                        ).wait()

                pltpu.async_copy(src, bufs[b], slds[b]).wait()
                for kk in range(GROUP):
                    pltpu.async_copy(
                        bufs[b].at[pl.ds(PAD - 8 * kk, C)], dsts[kk], ssts[b]
                    )
            return carry

        lax.fori_loop(0, K // NBUF, body, 0)
        src0, _ = plan(0)
        for b in range(NBUF):
            for _ in range(GROUP):
                pltpu.make_async_copy(
                    src0, bufs[b].at[pl.ds(0, C)], ssts[b]
                ).wait()

    return k(P)


def kernel(x, q_len, pe):
    n = x.shape[1]
    V = pe.shape[0]
    off = (V + 1) // 2 - 1
    return _sc_relpos_copy(pe, n, off)


# speedup vs baseline: 1.9305x; 1.0169x over previous
import functools

import jax
import jax.numpy as jnp
from jax import lax
from jax.experimental import pallas as pl
from jax.experimental.pallas import tpu as pltpu
from jax.experimental.pallas import tpu_sc as plsc


def _sc_relpos_copy(pe, n, off):
    V, D = pe.shape
    info = plsc.get_sparse_core_info()
    NC, NS = info.num_cores, info.num_subcores
    NW = NC * NS
    assert n % NW == 0
    rows_per_w = n // NW

    C = 256
    GROUP = 16
    PAD = 8 * (GROUP - 1)
    DS = 6
    D2 = D // DS
    NBUF = 2
    NCH = (n // C) * DS
    NGRP = rows_per_w // GROUP
    K = NGRP * NCH
    assert K % NBUF == 0

    lo = ((off - n + 1) // 8) * 8 - PAD
    P = jnp.stack(
        [lax.dynamic_slice_in_dim(pe, lo + d, 2 * n + PAD) for d in range(8)]
    )

    mesh = plsc.VectorSubcoreMesh(core_axis_name="c", subcore_axis_name="s")

    @functools.partial(
        pl.kernel,
        out_type=jax.ShapeDtypeStruct((n, n, D), jnp.float32),
        mesh=mesh,
        scratch_types=(
            [pltpu.VMEM((C + PAD, D2), jnp.float32)] * NBUF
            + [pltpu.SemaphoreType.DMA] * (2 * NBUF)
        ),
    )
    def k(p_hbm, out_hbm, *scratch):
        bufs = scratch[:NBUF]
        slds = scratch[NBUF:2 * NBUF]
        ssts = scratch[2 * NBUF:]
        wid = lax.axis_index("s") * NC + lax.axis_index("c")
        wpb = 8 * GROUP // rows_per_w
        blk = lax.div(wid, wpb) * (8 * GROUP)
        r0 = lax.rem(wid, wpb) * NGRP

        def plan(step):
            grp = lax.div(step, NCH)
            rem = lax.rem(step, NCH)
            jc = lax.div(rem, DS) * C
            h = lax.rem(rem, DS) * D2
            i = blk + r0 + grp
            s = off - i
            d = lax.rem(s, 8)
            a = pl.multiple_of(s - d - lo + jc - PAD, 8)
            hh = pl.multiple_of(h, 128)
            src = p_hbm.at[d, pl.ds(a, C + PAD), pl.ds(hh, D2)]
            dsts = [
                out_hbm.at[i + 8 * kk, pl.ds(jc, C), pl.ds(hh, D2)]
                for kk in range(GROUP)
            ]
            return src, dsts

        def body(g, carry):
            for b in range(NBUF):
                step = NBUF * g + b
                src, dsts = plan(step)

                @pl.when(g >= 1)
                def _():
                    for _ in range(GROUP):
                        pltpu.make_async_copy(
                            src, bufs[b].at[pl.ds(0, C)], ssts[b]
                        ).wait()

                pltpu.async_copy(src, bufs[b], slds[b]).wait()
                for kk in range(GROUP):
                    pltpu.async_copy(
                        bufs[b].at[pl.ds(PAD - 8 * kk, C)], dsts[kk], ssts[b]
                    )
            return carry

        lax.fori_loop(0, K // NBUF, body, 0)
        src0, _ = plan(0)
        for b in range(NBUF):
            for _ in range(GROUP):
                pltpu.make_async_copy(
                    src0, bufs[b].at[pl.ds(0, C)], ssts[b]
                ).wait()

    return k(P)


def kernel(x, q_len, pe):
    n = x.shape[1]
    V = pe.shape[0]
    off = (V + 1) // 2 - 1
    return _sc_relpos_copy(pe, n, off)
